# baseline (speedup 1.0000x reference)
import functools

import jax
import jax.numpy as jnp
from jax import lax
from jax.experimental import pallas as pl
from jax.experimental.pallas import tpu as pltpu
from jax.experimental.pallas import tpu_sc as plsc

D_MODEL = 64
NUM_CORES = 2
NUM_SUBCORES = 16
NW = NUM_CORES * NUM_SUBCORES
LANE = 128
NBUF = 5
D8 = D_MODEL // 8


@functools.partial(jax.jit, static_argnames=("bsz", "seq"))
def _emb_lookup(table, xt, bsz, seq):
    mesh = plsc.VectorSubcoreMesh(
        core_axis_name="c", subcore_axis_name="s",
        num_cores=NUM_CORES, num_subcores=NUM_SUBCORES)

    @functools.partial(
        pl.kernel,
        out_type=jax.ShapeDtypeStruct(
            (seq, D8, bsz // LANE, 8 * LANE), jnp.float32),
        mesh=mesh,
        scratch_types=[
            pltpu.VMEM((seq, LANE), jnp.int32),
            pltpu.VMEM((NBUF, LANE, D_MODEL), jnp.float32),
            pltpu.VMEM((NBUF, D_MODEL * LANE), jnp.float32),
            pltpu.VMEM((D_MODEL, 16), jnp.int32),
            pltpu.SemaphoreType.DMA,
            pltpu.SemaphoreType.DMA((NBUF,)),
            pltpu.SemaphoreType.DMA((NBUF,)),
        ],
        compiler_params=pltpu.CompilerParams(
            use_tc_tiling_on_sc=False, needs_layout_passes=False),
    )
    def k(table_hbm, xt_hbm, out_hbm, idx_t, rows_v, trans_v, dsel_ref,
          isem, gsems, osems):
        wid = lax.axis_index("s") * NUM_CORES + lax.axis_index("c")
        base = wid * LANE

        cp = pltpu.make_async_copy(
            xt_hbm.at[:, pl.ds(base, LANE)], idx_t, isem)
        cp.start()
        cp.wait()

        def g_copy(t, s):
            return pltpu.make_async_copy(
                table_hbm.at[idx_t.at[t]], rows_v.at[s], gsems.at[s])

        def o_copies(t, s):
            return [
                pltpu.make_async_copy(
                    trans_v.at[s, pl.ds(j * LANE * 8, LANE * 8)],
                    out_hbm.at[t, j, wid], osems.at[s])
                for j in range(D8)
            ]

        iota = lax.broadcasted_iota(jnp.int32, (16,), 0)
        rowsel = [iota + c * 16 for c in range(LANE // 16)]

        for d in range(D_MODEL):
            dsel_ref[d, :] = jnp.full((16,), d, jnp.int32)

        for s in range(NBUF):
            g_copy(s, s).start()

        n_rounds = seq // NBUF

        def round_body(r, carry):
            for s in range(NBUF):
                g_copy(r * NBUF + s, s).wait()

            for s in range(NBUF):
                for d in range(D_MODEL):
                    dsel = dsel_ref[d, :]
                    vs = [
                        plsc.load_gather(
                            rows_v.at[s], [rowsel[c], dsel])
                        for c in range(LANE // 16)
                    ]
                    for c in range(LANE // 16):
                        trans_v[s, pl.ds(d * LANE + c * 16, 16)] = vs[c]

            for s in range(NBUF):
                for c in o_copies(r * NBUF + s, s):
                    c.start()

            for s in range(NBUF):
                t = r * NBUF + s
                for c in o_copies(t, s):
                    c.wait()
                tn = t + NBUF

                @pl.when(tn < seq)
                def _():
                    g_copy(tn, s).start()

            return carry

        lax.fori_loop(0, n_rounds, round_body, 0)

    return k(table, xt)


def kernel(x, table):
    bsz, seq = x.shape
    out4 = _emb_lookup(table, x.T, bsz, seq)
    out5 = out4.reshape(seq, D8, bsz // LANE, 8, LANE)
    return out5.transpose(2, 4, 0, 1, 3).reshape(bsz, seq, D_MODEL)


# speedup vs baseline: 1.6817x; 1.6817x over previous
import functools

import jax
import jax.numpy as jnp
from jax import lax
from jax.experimental import pallas as pl
from jax.experimental.pallas import tpu as pltpu
from jax.experimental.pallas import tpu_sc as plsc

D_MODEL = 64
NUM_CORES = 2
NUM_SUBCORES = 16
NW = NUM_CORES * NUM_SUBCORES
NBUF = 4


@functools.partial(jax.jit, static_argnames=("bsz", "seq"))
def _emb_lookup(table, x, bsz, seq):
    mesh = plsc.VectorSubcoreMesh(
        core_axis_name="c", subcore_axis_name="s",
        num_cores=NUM_CORES, num_subcores=NUM_SUBCORES)
    rows_per_w = bsz // NW

    @functools.partial(
        pl.kernel,
        out_type=jax.ShapeDtypeStruct((bsz, seq, D_MODEL), jnp.float32),
        mesh=mesh,
        scratch_types=[
            pltpu.VMEM((rows_per_w, seq), jnp.int32),
            pltpu.VMEM((NBUF, seq, D_MODEL), jnp.float32),
            pltpu.SemaphoreType.DMA,
            pltpu.SemaphoreType.DMA((NBUF,)),
            pltpu.SemaphoreType.DMA((NBUF,)),
        ],
        compiler_params=pltpu.CompilerParams(use_tc_tiling_on_sc=False),
    )
    def k(table_hbm, x_hbm, out_hbm, idx_v, rows_v, isem, gsems, osems):
        wid = lax.axis_index("s") * NUM_CORES + lax.axis_index("c")
        base = wid * rows_per_w

        cp = pltpu.make_async_copy(
            x_hbm.at[pl.ds(base, rows_per_w)], idx_v, isem)
        cp.start()
        cp.wait()

        def g_copy(j, b):
            return pltpu.make_async_copy(
                table_hbm.at[idx_v.at[j]], rows_v.at[b], gsems.at[b])

        def o_copy(j, b):
            return pltpu.make_async_copy(
                rows_v.at[b], out_hbm.at[base + j], osems.at[b])

        for b in range(NBUF):
            g_copy(b, b).start()

        n_rounds = rows_per_w // NBUF

        def round_body(r, carry):
            for b in range(NBUF):
                j = r * NBUF + b
                g_copy(j, b).wait()
                o_copy(j, b).start()
            for b in range(NBUF):
                j = r * NBUF + b
                o_copy(j, b).wait()
                jn = j + NBUF

                @pl.when(jn < rows_per_w)
                def _():
                    g_copy(jn, b).start()

            return carry

        lax.fori_loop(0, n_rounds, round_body, 0)

    return k(table, x)


def kernel(x, table):
    bsz, seq = x.shape
    return _emb_lookup(table, x, bsz, seq)
